# baseline (speedup 1.0000x reference)
import functools

import jax
import jax.numpy as jnp
from jax import lax
from jax.experimental import pallas as pl
from jax.experimental.pallas import tpu as pltpu
from jax.experimental.pallas import tpu_sc as plsc

N = 16384
F = 128
H = 50
H2 = H // 2
G = 100000
GT = 800
HPAD = 64

NC = 2
NS = 16
L = 16
NW = NC * NS
BPW = N // NW
BLK = 1024
IR = 32
PB = 6400

_SELU_ALPHA = 1.6732632423543772848170429916717
_SELU_SCALE = 1.0507009873554804934193349852946


def _pack_pairs(rows2):
    k2, m = rows2.shape
    t3 = rows2.astype(jnp.bfloat16).reshape(k2 // 2, 2, m)
    lo = lax.bitcast_convert_type(t3[:, 0, :], jnp.uint16).astype(jnp.uint32)
    hi = lax.bitcast_convert_type(t3[:, 1, :], jnp.uint16).astype(jnp.uint32)
    return lax.bitcast_convert_type(lo | (hi << 16), jnp.int32)


def _tc_body(x_ref, w_ref, b_ref, g_ref, t_ref, act_ref, idx_ref, tab_ref):
    pre = lax.dot_general(w_ref[...], x_ref[...], (((0,), (1,)), ((), ())),
                          preferred_element_type=jnp.float32)
    pre = pre + b_ref[...]
    act = _SELU_SCALE * jnp.where(
        pre > 0, pre, _SELU_ALPHA * (jnp.exp(pre) - 1.0))
    actp = _pack_pairs(act[:H, :])
    actp = jnp.concatenate(
        [actp, jnp.zeros((IR - H2, BLK), jnp.int32)], axis=0)
    act_ref[...] = actp.reshape(IR, BLK // 128, 128)

    g2 = g_ref[0, 0, :].reshape(2, 1, BPW)
    base2 = (g2 >> 7) * (IR * 128) + (g2 & 127)
    hh = lax.broadcasted_iota(jnp.int32, (1, IR, 1), 1) * 128
    idx_ref[...] = (base2 + hh).reshape(2 * IR, BPW).reshape(
        2 * IR * BPW // 128, 128)

    packed = _pack_pairs(t_ref[...])
    full = jnp.concatenate(
        [packed, jnp.zeros((IR - H2, PB), jnp.int32)], axis=0)
    tab_ref[...] = jnp.swapaxes(full.reshape(IR, PB // 128, 128), 0, 1)


def _tc_fused(x, w1p, b1p, gene3, table):
    return pl.pallas_call(
        _tc_body,
        grid=(N // BLK,),
        in_specs=[
            pl.BlockSpec((BLK, F), lambda i: (i, 0)),
            pl.BlockSpec((F, HPAD), lambda i: (0, 0)),
            pl.BlockSpec((HPAD, 1), lambda i: (0, 0)),
            pl.BlockSpec((1, 1, BLK), lambda i: (i, 0, 0)),
            pl.BlockSpec((H, PB), lambda i: (0, i)),
        ],
        out_specs=[
            pl.BlockSpec((IR, BLK // 128, 128), lambda i: (0, i, 0)),
            pl.BlockSpec((2 * IR * BPW // 128, 128), lambda i: (i, 0)),
            pl.BlockSpec((PB // 128, IR, 128), lambda i: (i, 0, 0)),
        ],
        out_shape=[
            jax.ShapeDtypeStruct((IR, N // 128, 128), jnp.int32),
            jax.ShapeDtypeStruct((NW * IR * BPW // 128, 128), jnp.int32),
            jax.ShapeDtypeStruct((GT, IR, 128), jnp.int32),
        ],
    )(x, w1p, b1p, gene3, table)


_sc_mesh = plsc.VectorSubcoreMesh(
    core_axis_name="c", subcore_axis_name="s", num_cores=NC, num_subcores=NS)


@functools.partial(
    pl.kernel,
    out_type=(
        jax.ShapeDtypeStruct((N,), jnp.float32),
        jax.ShapeDtypeStruct((N,), jnp.float32),
        jax.ShapeDtypeStruct((N,), jnp.float32),
    ),
    mesh=_sc_mesh,
    scratch_types=[
        pltpu.VMEM((BPW,), jnp.int32),
        pltpu.VMEM((BPW,), jnp.float32),
        pltpu.VMEM((BPW,), jnp.float32),
        pltpu.VMEM((BPW,), jnp.float32),
        pltpu.SemaphoreType.DMA,
    ],
)
def _sc_scalar(gene_hbm, bias_hbm, phi_hbm, pz_hbm,
               bias_out, phi_out, pz_out,
               idx_v, bias_v, phi_v, pz_v, sem):
    wid = lax.axis_index("s") * NC + lax.axis_index("c")
    base = wid * BPW
    pltpu.sync_copy(gene_hbm.at[pl.ds(base, BPW)], idx_v)
    copies = [
        pltpu.async_copy(bias_hbm.at[idx_v], bias_v, sem),
        pltpu.async_copy(phi_hbm.at[idx_v], phi_v, sem),
        pltpu.async_copy(pz_hbm.at[idx_v], pz_v, sem),
    ]
    for c in copies:
        c.wait()
    pltpu.sync_copy(bias_v, bias_out.at[pl.ds(base, BPW)])
    pltpu.sync_copy(phi_v, phi_out.at[pl.ds(base, BPW)])
    pltpu.sync_copy(pz_v, pz_out.at[pl.ds(base, BPW)])


@functools.partial(
    pl.kernel,
    out_type=jax.ShapeDtypeStruct((N,), jnp.float32),
    mesh=_sc_mesh,
    scratch_types=[
        pltpu.VMEM((H2 * BPW,), jnp.int32),
        pltpu.VMEM((H2 * BPW,), jnp.int32),
        pltpu.VMEM((H2 * BPW,), jnp.int32),
        pltpu.VMEM((BPW,), jnp.float32),
        pltpu.VMEM((BPW,), jnp.float32),
        pltpu.SemaphoreType.DMA,
        pltpu.SemaphoreType.DMA,
    ],
)
def _sc_gather_dot(idx_hbm, act_hbm, table_hbm, bias_hbm,
                   mu_out,
                   idx_v, a_v, w_v, bias_v, mu_v,
                   sem_w, sem_a):
    wid = lax.axis_index("s") * NC + lax.axis_index("c")
    base = wid * BPW

    pltpu.sync_copy(idx_hbm.at[pl.ds(wid * IR * BPW, H2 * BPW)], idx_v)

    w_copies = []
    a_copies = []
    for h2 in range(H2):
        w_copies.append(pltpu.async_copy(
            table_hbm.at[idx_v.at[pl.ds(h2 * BPW, BPW)]],
            w_v.at[pl.ds(h2 * BPW, BPW)], sem_w))
        a_copies.append(pltpu.async_copy(
            act_hbm.at[pl.ds(h2 * N + base, BPW)],
            a_v.at[pl.ds(h2 * BPW, BPW)], sem_a))
    bias_copy = pltpu.async_copy(bias_hbm.at[pl.ds(base, BPW)], bias_v, sem_a)

    mask = jnp.full((L,), -65536, jnp.int32)
    for h2 in range(H2):
        w_copies[h2].wait()
        a_copies[h2].wait()

        def body(j, _):
            sl = pl.ds(h2 * BPW + j * L, L)
            wi = w_v[sl]
            ai = a_v[sl]
            wlo = lax.bitcast_convert_type(wi << 16, jnp.float32)
            whi = lax.bitcast_convert_type(wi & mask, jnp.float32)
            alo = lax.bitcast_convert_type(ai << 16, jnp.float32)
            ahi = lax.bitcast_convert_type(ai & mask, jnp.float32)
            prod = wlo * alo + whi * ahi
            osl = pl.ds(j * L, L)
            if h2 == 0:
                mu_v[osl] = prod
            else:
                mu_v[osl] = mu_v[osl] + prod
            return 0

        lax.fori_loop(0, BPW // L, body, 0)

    bias_copy.wait()

    def bbody(j, _):
        osl = pl.ds(j * L, L)
        mu_v[osl] = mu_v[osl] + bias_v[osl]
        return 0

    lax.fori_loop(0, BPW // L, bbody, 0)

    pltpu.sync_copy(mu_v, mu_out.at[pl.ds(base, BPW)])


def kernel(gene_index_tensor_n, cell_index_tensor_n, cell_features_nf,
           total_obs_reads_per_cell_tensor_n, downsampling_rate_tensor_n,
           W1, b1, readout_weight_hg, readout_bias_g,
           log_phi_e_hi_g, logit_p_zero_e_hi_g):
    del cell_index_tensor_n, total_obs_reads_per_cell_tensor_n
    del downsampling_rate_tensor_n
    gene_i32 = gene_index_tensor_n.astype(jnp.int32)
    gene3 = gene_i32.reshape(N // BLK, 1, BLK)
    w1p = jnp.pad(W1, ((0, 0), (0, HPAD - H)))
    b1p = jnp.pad(b1, (0, HPAD - H)).reshape(HPAD, 1)
    act3, idx4, tab3 = _tc_fused(cell_features_nf, w1p, b1p, gene3,
                                 readout_weight_hg)
    bias_n, phi, pz = _sc_scalar(gene_i32, readout_bias_g,
                                 log_phi_e_hi_g, logit_p_zero_e_hi_g)
    mu = _sc_gather_dot(idx4.reshape(NW * IR * BPW),
                        act3.reshape(IR * N),
                        tab3.reshape(GT * IR * 128), bias_n)
    return mu, phi, pz


# speedup vs baseline: 1.2712x; 1.2712x over previous
import functools

import jax
import jax.numpy as jnp
from jax import lax
from jax.experimental import pallas as pl
from jax.experimental.pallas import tpu as pltpu
from jax.experimental.pallas import tpu_sc as plsc

N = 16384
F = 128
H = 50
H2 = H // 2
G = 100000
GT = 800
HPAD = 64

NC = 2
NS = 16
L = 16
NW = NC * NS
BPW = N // NW
BLK = 1024
IR = 32
PB = 6400

_SELU_ALPHA = 1.6732632423543772848170429916717
_SELU_SCALE = 1.0507009873554804934193349852946


def _rne16(u):
    return u + 0x7FFF + ((u >> 16) & 1)


def _pack_pairs(rows2):
    k2, m = rows2.shape
    u3 = lax.bitcast_convert_type(rows2, jnp.int32).reshape(k2 // 2, 2, m)
    lo = lax.shift_right_logical(_rne16(u3[:, 0, :]), 16)
    hi = _rne16(u3[:, 1, :]) & jnp.int32(-65536)
    return lo | hi


def _tc_body(x_ref, w_ref, b_ref, g_ref, t_ref, act_ref, idx_ref, tab_ref):
    pre = lax.dot_general(w_ref[...], x_ref[...], (((0,), (1,)), ((), ())),
                          preferred_element_type=jnp.float32)
    pre = pre + b_ref[...]
    act = _SELU_SCALE * jnp.where(
        pre > 0, pre, _SELU_ALPHA * (jnp.exp(pre) - 1.0))
    actp = _pack_pairs(act[:H, :])
    actp = jnp.concatenate(
        [actp, jnp.zeros((IR - H2, BLK), jnp.int32)], axis=0)
    act_ref[...] = actp.reshape(IR, BLK // 128, 128)

    g2 = g_ref[0, 0, :].reshape(2, 1, BPW)
    base2 = (g2 >> 7) * (IR * 128) + (g2 & 127)
    hh = lax.broadcasted_iota(jnp.int32, (1, IR, 1), 1) * 128
    idx_ref[...] = (base2 + hh).reshape(2 * IR, BPW).reshape(
        2 * IR * BPW // 128, 128)

    packed = _pack_pairs(t_ref[...])
    full = jnp.concatenate(
        [packed, jnp.zeros((IR - H2, PB), jnp.int32)], axis=0)
    tab_ref[...] = jnp.swapaxes(full.reshape(IR, PB // 128, 128), 0, 1)


def _tc_fused(x, w1p, b1p, gene3, table):
    return pl.pallas_call(
        _tc_body,
        grid=(N // BLK,),
        in_specs=[
            pl.BlockSpec((BLK, F), lambda i: (i, 0)),
            pl.BlockSpec((F, HPAD), lambda i: (0, 0)),
            pl.BlockSpec((HPAD, 1), lambda i: (0, 0)),
            pl.BlockSpec((1, 1, BLK), lambda i: (i, 0, 0)),
            pl.BlockSpec((H, PB), lambda i: (0, i)),
        ],
        out_specs=[
            pl.BlockSpec((IR, BLK // 128, 128), lambda i: (0, i, 0)),
            pl.BlockSpec((2 * IR * BPW // 128, 128), lambda i: (i, 0)),
            pl.BlockSpec((PB // 128, IR, 128), lambda i: (i, 0, 0)),
        ],
        out_shape=[
            jax.ShapeDtypeStruct((IR, N // 128, 128), jnp.int32),
            jax.ShapeDtypeStruct((NW * IR * BPW // 128, 128), jnp.int32),
            jax.ShapeDtypeStruct((GT, IR, 128), jnp.int32),
        ],
    )(x, w1p, b1p, gene3, table)


_sc_mesh = plsc.VectorSubcoreMesh(
    core_axis_name="c", subcore_axis_name="s", num_cores=NC, num_subcores=NS)


@functools.partial(
    pl.kernel,
    out_type=(
        jax.ShapeDtypeStruct((N,), jnp.float32),
        jax.ShapeDtypeStruct((N,), jnp.float32),
        jax.ShapeDtypeStruct((N,), jnp.float32),
    ),
    mesh=_sc_mesh,
    scratch_types=[
        pltpu.VMEM((BPW,), jnp.int32),
        pltpu.VMEM((BPW,), jnp.float32),
        pltpu.VMEM((BPW,), jnp.float32),
        pltpu.VMEM((BPW,), jnp.float32),
        pltpu.SemaphoreType.DMA,
    ],
)
def _sc_scalar(gene_hbm, bias_hbm, phi_hbm, pz_hbm,
               bias_out, phi_out, pz_out,
               idx_v, bias_v, phi_v, pz_v, sem):
    wid = lax.axis_index("s") * NC + lax.axis_index("c")
    base = wid * BPW
    pltpu.sync_copy(gene_hbm.at[pl.ds(base, BPW)], idx_v)
    copies = [
        pltpu.async_copy(bias_hbm.at[idx_v], bias_v, sem),
        pltpu.async_copy(phi_hbm.at[idx_v], phi_v, sem),
        pltpu.async_copy(pz_hbm.at[idx_v], pz_v, sem),
    ]
    for c in copies:
        c.wait()
    pltpu.sync_copy(bias_v, bias_out.at[pl.ds(base, BPW)])
    pltpu.sync_copy(phi_v, phi_out.at[pl.ds(base, BPW)])
    pltpu.sync_copy(pz_v, pz_out.at[pl.ds(base, BPW)])


@functools.partial(
    pl.kernel,
    out_type=jax.ShapeDtypeStruct((N,), jnp.float32),
    mesh=_sc_mesh,
    scratch_types=[
        pltpu.VMEM((H2 * BPW,), jnp.int32),
        pltpu.VMEM((H2 * BPW,), jnp.int32),
        pltpu.VMEM((H2 * BPW,), jnp.int32),
        pltpu.VMEM((BPW,), jnp.float32),
        pltpu.VMEM((BPW,), jnp.float32),
        pltpu.SemaphoreType.DMA,
        pltpu.SemaphoreType.DMA,
    ],
)
def _sc_gather_dot(idx_hbm, act_hbm, table_hbm, bias_hbm,
                   mu_out,
                   idx_v, a_v, w_v, bias_v, mu_v,
                   sem_w, sem_a):
    wid = lax.axis_index("s") * NC + lax.axis_index("c")
    base = wid * BPW

    pltpu.sync_copy(idx_hbm.at[pl.ds(wid * IR * BPW, H2 * BPW)], idx_v)

    w_copies = []
    a_copies = []
    for h2 in range(H2):
        w_copies.append(pltpu.async_copy(
            table_hbm.at[idx_v.at[pl.ds(h2 * BPW, BPW)]],
            w_v.at[pl.ds(h2 * BPW, BPW)], sem_w))
        a_copies.append(pltpu.async_copy(
            act_hbm.at[pl.ds(h2 * N + base, BPW)],
            a_v.at[pl.ds(h2 * BPW, BPW)], sem_a))
    bias_copy = pltpu.async_copy(bias_hbm.at[pl.ds(base, BPW)], bias_v, sem_a)

    mask = jnp.full((L,), -65536, jnp.int32)
    for h2 in range(H2):
        w_copies[h2].wait()
        a_copies[h2].wait()

        def body(j, _):
            sl = pl.ds(h2 * BPW + j * L, L)
            wi = w_v[sl]
            ai = a_v[sl]
            wlo = lax.bitcast_convert_type(wi << 16, jnp.float32)
            whi = lax.bitcast_convert_type(wi & mask, jnp.float32)
            alo = lax.bitcast_convert_type(ai << 16, jnp.float32)
            ahi = lax.bitcast_convert_type(ai & mask, jnp.float32)
            prod = wlo * alo + whi * ahi
            osl = pl.ds(j * L, L)
            if h2 == 0:
                mu_v[osl] = prod
            else:
                mu_v[osl] = mu_v[osl] + prod
            return 0

        lax.fori_loop(0, BPW // L, body, 0)

    bias_copy.wait()

    def bbody(j, _):
        osl = pl.ds(j * L, L)
        mu_v[osl] = mu_v[osl] + bias_v[osl]
        return 0

    lax.fori_loop(0, BPW // L, bbody, 0)

    pltpu.sync_copy(mu_v, mu_out.at[pl.ds(base, BPW)])


def kernel(gene_index_tensor_n, cell_index_tensor_n, cell_features_nf,
           total_obs_reads_per_cell_tensor_n, downsampling_rate_tensor_n,
           W1, b1, readout_weight_hg, readout_bias_g,
           log_phi_e_hi_g, logit_p_zero_e_hi_g):
    del cell_index_tensor_n, total_obs_reads_per_cell_tensor_n
    del downsampling_rate_tensor_n
    gene_i32 = gene_index_tensor_n.astype(jnp.int32)
    gene3 = gene_i32.reshape(N // BLK, 1, BLK)
    w1p = jnp.pad(W1, ((0, 0), (0, HPAD - H)))
    b1p = jnp.pad(b1, (0, HPAD - H)).reshape(HPAD, 1)
    act3, idx4, tab3 = _tc_fused(cell_features_nf, w1p, b1p, gene3,
                                 readout_weight_hg)
    bias_n, phi, pz = _sc_scalar(gene_i32, readout_bias_g,
                                 log_phi_e_hi_g, logit_p_zero_e_hi_g)
    mu = _sc_gather_dot(idx4.reshape(NW * IR * BPW),
                        act3.reshape(IR * N),
                        tab3.reshape(GT * IR * 128), bias_n)
    return mu, phi, pz
